# Initial kernel scaffold; baseline (speedup 1.0000x reference)
import jax
import jax.numpy as jnp
from jax.experimental import pallas as pl


def kernel(users_feat, exercises_feat, edge_index):
    raise NotImplementedError("write your pallas kernel here")


# speedup vs baseline: 10.6308x; 10.6308x over previous
import functools

import jax
import jax.numpy as jnp
from jax import lax
from jax.experimental import pallas as pl
from jax.experimental.pallas import tpu as pltpu
from jax.experimental.pallas import tpu_sc as plsc

U = 25000
NI = 25000
D = 64
E = 800000
NUM_LAYERS = 2

NC = 2
NS = 16
CH = 96
NRING = 4
EDGES_PER_TILE = 50688
E_PAD = EDGES_PER_TILE * NS
NCHUNK = E_PAD // CH
CHUNKS_PER_TILE = NCHUNK // NS
NPAD = 25088
ROWS_PER_TILE = NPAD // NS
DUMMY = 25000

_mesh = plsc.VectorSubcoreMesh(
    core_axis_name="c", subcore_axis_name="s", num_cores=NC, num_subcores=NS
)
_sc_params = pltpu.CompilerParams(use_tc_tiling_on_sc=False)


@functools.partial(
    pl.kernel,
    out_type=jax.ShapeDtypeStruct((NC, NPAD), jnp.float32),
    mesh=_mesh,
    compiler_params=_sc_params,
    scratch_types=[
        pltpu.VMEM_SHARED((NPAD,), jnp.float32),
        pltpu.VMEM((NRING, CH), jnp.int32),
        pltpu.VMEM((CH,), jnp.float32),
        pltpu.SemaphoreType.DMA((NRING,)),
    ],
)
def _degree_kernel(sidx_h, zeros1_h, deg_out, deg_sh, sidx_v, ones_v, sems):
    c = lax.axis_index("c")
    s = lax.axis_index("s")
    base = s * CHUNKS_PER_TILE

    @pl.when(s == 0)
    def _():
        pltpu.sync_copy(zeros1_h, deg_sh)

    for q in range(CH // 16):
        ones_v[pl.ds(q * 16, 16)] = jnp.ones((16,), jnp.float32)
    plsc.subcore_barrier()

    def fire(r, g):
        pltpu.sync_copy(sidx_h.at[c].at[base + g], sidx_v.at[r])
        pltpu.async_copy(ones_v, deg_sh.at[sidx_v.at[r]], sems.at[r], add=True)

    def drain(r):
        pltpu.make_async_copy(zeros1_h.at[pl.ds(0, CH)], ones_v,
                              sems.at[r]).wait()

    for r in range(NRING):
        fire(r, r)

    @pl.loop(0, CHUNKS_PER_TILE // NRING - 1)
    def _(q):
        for r in range(NRING):
            drain(r)
            fire(r, NRING * q + r + NRING)

    for r in range(NRING):
        drain(r)

    plsc.subcore_barrier()

    @pl.when(s == 0)
    def _():
        pltpu.sync_copy(deg_sh, deg_out.at[c])


@functools.partial(
    pl.kernel,
    out_type=jax.ShapeDtypeStruct((NC, NPAD, D), jnp.float32),
    mesh=_mesh,
    compiler_params=_sc_params,
    scratch_types=[
        pltpu.VMEM_SHARED((NPAD, D), jnp.float32),
        pltpu.VMEM((NRING, CH, D), jnp.float32),
        pltpu.VMEM((NRING, CH), jnp.int32),
        pltpu.VMEM((NRING, CH), jnp.int32),
        pltpu.SemaphoreType.DMA((NRING,)),
    ],
)
def _spmm_kernel(tab_h, gidx_h, sidx_h, zeros2_h, msg_out,
                 acc_sh, rows_v, gidx_v, sidx_v, sems):
    c = lax.axis_index("c")
    s = lax.axis_index("s")
    base = s * CHUNKS_PER_TILE
    rbase = s * ROWS_PER_TILE

    pltpu.sync_copy(zeros2_h.at[pl.ds(rbase, ROWS_PER_TILE)],
                    acc_sh.at[pl.ds(rbase, ROWS_PER_TILE)])
    plsc.subcore_barrier()

    def fire(r, g):
        pltpu.sync_copy(gidx_h.at[c].at[base + g], gidx_v.at[r])
        pltpu.sync_copy(sidx_h.at[c].at[base + g], sidx_v.at[r])
        pltpu.async_copy(tab_h.at[c].at[gidx_v.at[r]], rows_v.at[r],
                         sems.at[r])

    def drain(r):
        pltpu.make_async_copy(zeros2_h.at[pl.ds(0, CH)], rows_v.at[r],
                              sems.at[r]).wait()
        pltpu.sync_copy(rows_v.at[r], acc_sh.at[sidx_v.at[r]], add=True)

    for r in range(NRING):
        fire(r, r)

    @pl.loop(0, CHUNKS_PER_TILE // NRING - 1)
    def _(q):
        for r in range(NRING):
            drain(r)
            fire(r, NRING * q + r + NRING)

    for r in range(NRING):
        drain(r)

    plsc.subcore_barrier()
    pltpu.sync_copy(acc_sh.at[pl.ds(rbase, ROWS_PER_TILE)],
                    msg_out.at[c].at[pl.ds(rbase, ROWS_PER_TILE)])


def _prep_body(feats_ref, deg_ref, tab_ref, inv_ref):
    deg = deg_ref[0, 0, :]
    inv = 1.0 / (jnp.sqrt(deg) + 1e-8)
    inv_ref[0, 0, :] = inv
    tab_ref[0] = feats_ref[0] * inv[:, None]


RB = 512
NRB = NPAD // RB


def _tc_prep(feats0, deg):
    return pl.pallas_call(
        _prep_body,
        grid=(2, NRB),
        in_specs=[
            pl.BlockSpec((1, RB, D), lambda cc, b: (1 - cc, b, 0)),
            pl.BlockSpec((1, 1, RB), lambda cc, b: (1 - cc, 0, b)),
        ],
        out_specs=[
            pl.BlockSpec((1, RB, D), lambda cc, b: (cc, b, 0)),
            pl.BlockSpec((1, 1, RB), lambda cc, b: (1 - cc, 0, b)),
        ],
        out_shape=[
            jax.ShapeDtypeStruct((NC, NPAD, D), jnp.float32),
            jax.ShapeDtypeStruct((NC, 1, NPAD), jnp.float32),
        ],
    )(feats0, deg)


def _mid_body(m_ref, inv_ref, tab_ref):
    inv = inv_ref[0, 0, :]
    tab_ref[0] = m_ref[0] * (inv * inv)[:, None]


def _tc_mid(m1, inv):
    return pl.pallas_call(
        _mid_body,
        grid=(2, NRB),
        in_specs=[
            pl.BlockSpec((1, RB, D), lambda cc, b: (1 - cc, b, 0)),
            pl.BlockSpec((1, 1, RB), lambda cc, b: (1 - cc, 0, b)),
        ],
        out_specs=pl.BlockSpec((1, RB, D), lambda cc, b: (cc, b, 0)),
        out_shape=jax.ShapeDtypeStruct((NC, NPAD, D), jnp.float32),
    )(m1, inv)


def _final_body(f_ref, m1_ref, m2_ref, inv_ref, out_ref):
    inv = inv_ref[0, 0, :]
    out_ref[0] = (f_ref[0] + (m1_ref[0] + m2_ref[0]) * inv[:, None]) * (
        1.0 / (NUM_LAYERS + 1))


def _tc_final(feats0, m1, m2, inv):
    return pl.pallas_call(
        _final_body,
        grid=(2, NRB),
        in_specs=[
            pl.BlockSpec((1, RB, D), lambda cc, b: (cc, b, 0)),
            pl.BlockSpec((1, RB, D), lambda cc, b: (cc, b, 0)),
            pl.BlockSpec((1, RB, D), lambda cc, b: (cc, b, 0)),
            pl.BlockSpec((1, 1, RB), lambda cc, b: (cc, 0, b)),
        ],
        out_specs=pl.BlockSpec((1, RB, D), lambda cc, b: (cc, b, 0)),
        out_shape=jax.ShapeDtypeStruct((NC, NPAD, D), jnp.float32),
    )(feats0, m1, m2, inv)


def kernel(users_feat, exercises_feat, edge_index):
    src = edge_index[0].astype(jnp.int32)
    dst = edge_index[1].astype(jnp.int32)
    pad = jnp.full((E_PAD - E,), DUMMY, dtype=jnp.int32)
    src_p = jnp.concatenate([src, pad])
    dst_p = jnp.concatenate([dst, pad])
    gidx = jnp.stack([dst_p, src_p]).reshape(NC, NCHUNK, CH)
    sidx = jnp.stack([src_p, dst_p]).reshape(NC, NCHUNK, CH)

    zpadrows = jnp.zeros((NPAD - U, D), dtype=jnp.float32)
    users_pad = jnp.concatenate([users_feat, zpadrows])
    ex_pad = jnp.concatenate([exercises_feat, zpadrows])
    feats0 = jnp.stack([users_pad, ex_pad])
    zeros1 = jnp.zeros((NPAD,), dtype=jnp.float32)
    zeros2 = jnp.zeros((NPAD, D), dtype=jnp.float32)

    deg = _degree_kernel(sidx, zeros1)
    deg3 = deg.reshape(NC, 1, NPAD)
    tab1, inv = _tc_prep(feats0, deg3)
    m1 = _spmm_kernel(tab1, gidx, sidx, zeros2)
    tab2 = _tc_mid(m1, inv)
    m2 = _spmm_kernel(tab2, gidx, sidx, zeros2)
    acc = _tc_final(feats0, m1, m2, inv)

    return jnp.concatenate([acc[0, :U], acc[1, :NI]], axis=0)


# speedup vs baseline: 15.0573x; 1.4164x over previous
import functools

import jax
import jax.numpy as jnp
from jax import lax
from jax.experimental import pallas as pl
from jax.experimental.pallas import tpu as pltpu
from jax.experimental.pallas import tpu_sc as plsc

U = 25000
NI = 25000
D = 64
E = 800000
NUM_LAYERS = 2

NC = 2
NS = 16
CH = 96
NRING = 4
IB = 8
LAG = 2
NB = 66
EDGES_PER_TILE = 50688
E_PAD = EDGES_PER_TILE * NS
NCHUNK = E_PAD // CH
CHUNKS_PER_TILE = NCHUNK // NS
NPAD = 25088
ROWS_PER_TILE = NPAD // NS
DUMMY = 25000

_mesh = plsc.VectorSubcoreMesh(
    core_axis_name="c", subcore_axis_name="s", num_cores=NC, num_subcores=NS
)
_sc_params = pltpu.CompilerParams(use_tc_tiling_on_sc=False)


@functools.partial(
    pl.kernel,
    out_type=jax.ShapeDtypeStruct((NC, NPAD), jnp.float32),
    mesh=_mesh,
    compiler_params=_sc_params,
    scratch_types=[
        pltpu.VMEM_SHARED((NPAD,), jnp.float32),
        pltpu.VMEM((2, IB, CH), jnp.int32),
        pltpu.VMEM((CH,), jnp.float32),
        pltpu.SemaphoreType.DMA((NRING,)),
    ],
)
def _degree_kernel(sidx_h, zeros1_h, deg_out, deg_sh, sidx_v, ones_v, ssems):
    c = lax.axis_index("c")
    s = lax.axis_index("s")
    base = s * CHUNKS_PER_TILE

    @pl.when(s == 0)
    def _():
        pltpu.sync_copy(zeros1_h, deg_sh)

    for q in range(CH // 16):
        ones_v[pl.ds(q * 16, 16)] = jnp.ones((16,), jnp.float32)
    plsc.subcore_barrier()

    def fire(pb, k):
        pltpu.async_copy(ones_v, deg_sh.at[sidx_v.at[pb].at[k]],
                         ssems.at[k % NRING], add=True)

    def swait(k):
        pltpu.make_async_copy(zeros1_h.at[pl.ds(0, CH)], ones_v,
                              ssems.at[k % NRING]).wait()

    pltpu.sync_copy(sidx_h.at[c].at[pl.ds(base, IB)], sidx_v.at[0])
    for k in range(IB):
        if k >= NRING:
            swait(k)
        fire(0, k)

    @pl.loop(1, NB)
    def _(b):
        p = lax.rem(b, 2)
        g0 = base + b * IB
        pltpu.sync_copy(sidx_h.at[c].at[pl.ds(g0, IB)], sidx_v.at[p])
        for k in range(IB):
            swait(k)
            fire(p, k)

    for k in range(NRING):
        swait(k)

    plsc.subcore_barrier()

    @pl.when(s == 0)
    def _():
        pltpu.sync_copy(deg_sh, deg_out.at[c])


@functools.partial(
    pl.kernel,
    out_type=jax.ShapeDtypeStruct((NC, NPAD, D), jnp.float32),
    mesh=_mesh,
    compiler_params=_sc_params,
    scratch_types=[
        pltpu.VMEM_SHARED((NPAD, D), jnp.float32),
        pltpu.VMEM((NRING, CH, D), jnp.float32),
        pltpu.VMEM((2, IB, CH), jnp.int32),
        pltpu.VMEM((2, IB, CH), jnp.int32),
        pltpu.SemaphoreType.DMA((NRING,)),
        pltpu.SemaphoreType.DMA((NRING,)),
    ],
)
def _spmm_kernel(tab_h, gidx_h, sidx_h, zeros2_h, msg_out,
                 acc_sh, rows_v, gidx_v, sidx_v, gsems, ssems):
    c = lax.axis_index("c")
    s = lax.axis_index("s")
    base = s * CHUNKS_PER_TILE
    rbase = s * ROWS_PER_TILE

    pltpu.sync_copy(zeros2_h.at[pl.ds(rbase, ROWS_PER_TILE)],
                    acc_sh.at[pl.ds(rbase, ROWS_PER_TILE)])
    plsc.subcore_barrier()

    def gfire(pb, k, g):
        pltpu.async_copy(tab_h.at[c].at[gidx_v.at[pb].at[k]],
                         rows_v.at[k % NRING], gsems.at[k % NRING])

    def gwait(k):
        pltpu.make_async_copy(zeros2_h.at[pl.ds(0, CH)], rows_v.at[k % NRING],
                              gsems.at[k % NRING]).wait()

    def sfire(pb, k):
        pltpu.async_copy(rows_v.at[k % NRING], acc_sh.at[sidx_v.at[pb].at[k]],
                         ssems.at[k % NRING], add=True)

    def swait(k):
        pltpu.make_async_copy(zeros2_h.at[pl.ds(0, CH)], rows_v.at[k % NRING],
                              ssems.at[k % NRING]).wait()

    pltpu.sync_copy(gidx_h.at[c].at[pl.ds(base, IB)], gidx_v.at[0])
    pltpu.sync_copy(sidx_h.at[c].at[pl.ds(base, IB)], sidx_v.at[0])
    for k in range(IB):
        if k >= NRING:
            swait(k)
        gfire(0, k, k)
        if k >= LAG:
            gwait(k - LAG)
            sfire(0, k - LAG)

    @pl.loop(1, NB)
    def _(b):
        p = lax.rem(b, 2)
        pm = 1 - p
        g0 = base + b * IB
        pltpu.sync_copy(gidx_h.at[c].at[pl.ds(g0, IB)], gidx_v.at[p])
        pltpu.sync_copy(sidx_h.at[c].at[pl.ds(g0, IB)], sidx_v.at[p])
        for k in range(IB):
            swait(k)
            gfire(p, k, k)
            kk = k - LAG
            if kk >= 0:
                gwait(kk)
                sfire(p, kk)
            else:
                gwait(kk + IB)
                sfire(pm, kk + IB)

    ptail = (NB - 1) % 2
    for k in range(IB - LAG, IB):
        gwait(k)
        sfire(ptail, k)
    for k in range(NRING):
        swait(k)

    plsc.subcore_barrier()
    pltpu.sync_copy(acc_sh.at[pl.ds(rbase, ROWS_PER_TILE)],
                    msg_out.at[c].at[pl.ds(rbase, ROWS_PER_TILE)])


def _prep_body(feats_ref, deg_ref, tab_ref, inv_ref):
    deg = deg_ref[0, 0, :]
    inv = 1.0 / (jnp.sqrt(deg) + 1e-8)
    inv_ref[0, 0, :] = inv
    tab_ref[0] = feats_ref[0] * inv[:, None]


RB = 512
NRB = NPAD // RB


def _tc_prep(feats0, deg):
    return pl.pallas_call(
        _prep_body,
        grid=(2, NRB),
        in_specs=[
            pl.BlockSpec((1, RB, D), lambda cc, b: (1 - cc, b, 0)),
            pl.BlockSpec((1, 1, RB), lambda cc, b: (1 - cc, 0, b)),
        ],
        out_specs=[
            pl.BlockSpec((1, RB, D), lambda cc, b: (cc, b, 0)),
            pl.BlockSpec((1, 1, RB), lambda cc, b: (1 - cc, 0, b)),
        ],
        out_shape=[
            jax.ShapeDtypeStruct((NC, NPAD, D), jnp.float32),
            jax.ShapeDtypeStruct((NC, 1, NPAD), jnp.float32),
        ],
    )(feats0, deg)


def _mid_body(m_ref, inv_ref, tab_ref):
    inv = inv_ref[0, 0, :]
    tab_ref[0] = m_ref[0] * (inv * inv)[:, None]


def _tc_mid(m1, inv):
    return pl.pallas_call(
        _mid_body,
        grid=(2, NRB),
        in_specs=[
            pl.BlockSpec((1, RB, D), lambda cc, b: (1 - cc, b, 0)),
            pl.BlockSpec((1, 1, RB), lambda cc, b: (1 - cc, 0, b)),
        ],
        out_specs=pl.BlockSpec((1, RB, D), lambda cc, b: (cc, b, 0)),
        out_shape=jax.ShapeDtypeStruct((NC, NPAD, D), jnp.float32),
    )(m1, inv)


def _final_body(f_ref, m1_ref, m2_ref, inv_ref, out_ref):
    inv = inv_ref[0, 0, :]
    out_ref[0] = (f_ref[0] + (m1_ref[0] + m2_ref[0]) * inv[:, None]) * (
        1.0 / (NUM_LAYERS + 1))


def _tc_final(feats0, m1, m2, inv):
    return pl.pallas_call(
        _final_body,
        grid=(2, NRB),
        in_specs=[
            pl.BlockSpec((1, RB, D), lambda cc, b: (cc, b, 0)),
            pl.BlockSpec((1, RB, D), lambda cc, b: (cc, b, 0)),
            pl.BlockSpec((1, RB, D), lambda cc, b: (cc, b, 0)),
            pl.BlockSpec((1, 1, RB), lambda cc, b: (cc, 0, b)),
        ],
        out_specs=pl.BlockSpec((1, RB, D), lambda cc, b: (cc, b, 0)),
        out_shape=jax.ShapeDtypeStruct((NC, NPAD, D), jnp.float32),
    )(feats0, m1, m2, inv)


def kernel(users_feat, exercises_feat, edge_index):
    src = edge_index[0].astype(jnp.int32)
    dst = edge_index[1].astype(jnp.int32)
    pad = jnp.full((E_PAD - E,), DUMMY, dtype=jnp.int32)
    src_p = jnp.concatenate([src, pad])
    dst_p = jnp.concatenate([dst, pad])
    gidx = jnp.stack([dst_p, src_p]).reshape(NC, NCHUNK, CH)
    sidx = jnp.stack([src_p, dst_p]).reshape(NC, NCHUNK, CH)

    zpadrows = jnp.zeros((NPAD - U, D), dtype=jnp.float32)
    users_pad = jnp.concatenate([users_feat, zpadrows])
    ex_pad = jnp.concatenate([exercises_feat, zpadrows])
    feats0 = jnp.stack([users_pad, ex_pad])
    zeros1 = jnp.zeros((NPAD,), dtype=jnp.float32)
    zeros2 = jnp.zeros((NPAD, D), dtype=jnp.float32)

    deg = _degree_kernel(sidx, zeros1)
    deg3 = deg.reshape(NC, 1, NPAD)
    tab1, inv = _tc_prep(feats0, deg3)
    m1 = _spmm_kernel(tab1, gidx, sidx, zeros2)
    tab2 = _tc_mid(m1, inv)
    m2 = _spmm_kernel(tab2, gidx, sidx, zeros2)
    acc = _tc_final(feats0, m1, m2, inv)

    return jnp.concatenate([acc[0, :U], acc[1, :NI]], axis=0)
